# baseline (speedup 1.0000x reference)
import jax
import jax.numpy as jnp
from jax import lax
from jax.experimental import pallas as pl

N = 1000000
ROWS = 1000
COLS = 1000
BIG = 2**30


def _onehot_body(x_ref, out_ref):
    xv = x_ref[...]
    m = jnp.max(xv)
    rows = lax.broadcasted_iota(jnp.int32, (ROWS, COLS), 0)
    cols = lax.broadcasted_iota(jnp.int32, (ROWS, COLS), 1)
    lin = rows * COLS + cols
    cand = jnp.where(xv == m, lin, BIG)
    idx = jnp.min(cand)
    out_ref[...] = jnp.where(lin == idx, 1, 0).astype(jnp.int32)


def kernel(x):
    x2 = x.reshape(ROWS, COLS)
    out = pl.pallas_call(
        _onehot_body,
        out_shape=jax.ShapeDtypeStruct((ROWS, COLS), jnp.int32),
    )(x2)
    return out.reshape(N)
